# Initial kernel scaffold; baseline (speedup 1.0000x reference)
import jax
import jax.numpy as jnp
from jax.experimental import pallas as pl


def kernel(x_cat, x_cont, tables, gamma, beta, W0, b0, W1, b1, Wout, bout):
    raise NotImplementedError("write your pallas kernel here")


# speedup vs baseline: 2.2364x; 2.2364x over previous
import functools

import jax
import jax.numpy as jnp
from jax import lax
from jax.experimental import pallas as pl
from jax.experimental.pallas import tpu as pltpu
from jax.experimental.pallas import tpu_sc as plsc

_B, _F, _V, _D, _C = 16384, 26, 100000, 16, 13
_H0, _H1 = 512, 256
_E = _F * _D
_EPS = 1e-5
_CP = 128

_NC, _NS = 2, 16
_NW = _NC * _NS
_ROWS = _B * _F
_GROUPS = _ROWS // 128
_GPW = _GROUPS // _NW
_KG = 8
_NCHUNK = _GPW // _KG


def _make_gather():
  mesh = plsc.VectorSubcoreMesh(core_axis_name="c", subcore_axis_name="s")

  @functools.partial(
      pl.kernel,
      mesh=mesh,
      out_type=jax.ShapeDtypeStruct((_GROUPS, 128, _D), jnp.float32),
      scratch_types=[
          pltpu.VMEM((_KG, 128), jnp.int32),
          pltpu.VMEM((_KG, 128, _D), jnp.float32),
          pltpu.SemaphoreType.DMA,
      ],
      compiler_params=pltpu.CompilerParams(use_tc_tiling_on_sc=False),
  )
  def gather_k(tab_hbm, idx_hbm, out_hbm, idx_v, rows_v, sem):
    wid = lax.axis_index("s") * _NC + lax.axis_index("c")
    g0 = wid * _GPW

    def chunk(c, carry):
      base = g0 + c * _KG
      pltpu.sync_copy(idx_hbm.at[pl.ds(base, _KG)], idx_v)
      cps = [
          pltpu.async_copy(tab_hbm.at[idx_v.at[j]], rows_v.at[j], sem)
          for j in range(_KG)
      ]
      for cp in cps:
        cp.wait()
      pltpu.sync_copy(rows_v, out_hbm.at[pl.ds(base, _KG)])
      return carry

    lax.fori_loop(0, _NCHUNK, chunk, 0)

  return gather_k


_gather = _make_gather()


def _mlp_body(xcf_ref, emb_ref, xc_ref, w0a_ref, w0b_ref, b0_ref, w1_ref,
              b1_ref, wo_ref, bo_ref, out_ref, stats_ref):
  @pl.when(pl.program_id(0) == 0)
  def _():
    xc = xcf_ref[...]
    m = jnp.mean(xc, axis=0, keepdims=True)
    v = jnp.mean(xc * xc, axis=0, keepdims=True) - m * m
    stats_ref[0:1, :] = m
    stats_ref[1:2, :] = lax.rsqrt(v + _EPS)

  m = stats_ref[0:1, :]
  rstd = stats_ref[1:2, :]
  x2 = (xc_ref[...] - m) * rstd
  h = lax.dot_general(emb_ref[...], w0a_ref[...],
                      (((1,), (1,)), ((), ())),
                      preferred_element_type=jnp.float32)
  h = h + lax.dot_general(x2, w0b_ref[...],
                          (((1,), (1,)), ((), ())),
                          preferred_element_type=jnp.float32)
  h = jnp.maximum(h + b0_ref[...], 0.0)
  h = lax.dot_general(h, w1_ref[...],
                      (((1,), (1,)), ((), ())),
                      preferred_element_type=jnp.float32)
  h = jnp.maximum(h + b1_ref[...], 0.0)
  out_ref[...] = jnp.sum(h * wo_ref[...], axis=1, keepdims=True) + bo_ref[...]


_BLK = 1024


def _mlp(emb, xc, w0a, w0b_eff, b0_eff, w1, b1r, wo_r, bo_r):
  grid = (_B // _BLK,)
  return pl.pallas_call(
      _mlp_body,
      grid=grid,
      in_specs=[
          pl.BlockSpec((_B, _CP), lambda i: (0, 0)),
          pl.BlockSpec((_BLK, _E), lambda i: (i, 0)),
          pl.BlockSpec((_BLK, _CP), lambda i: (i, 0)),
          pl.BlockSpec((_H0, _E), lambda i: (0, 0)),
          pl.BlockSpec((_H0, _CP), lambda i: (0, 0)),
          pl.BlockSpec((1, _H0), lambda i: (0, 0)),
          pl.BlockSpec((_H1, _H0), lambda i: (0, 0)),
          pl.BlockSpec((1, _H1), lambda i: (0, 0)),
          pl.BlockSpec((1, _H1), lambda i: (0, 0)),
          pl.BlockSpec((1, 1), lambda i: (0, 0)),
      ],
      out_specs=pl.BlockSpec((_BLK, 1), lambda i: (i, 0)),
      out_shape=jax.ShapeDtypeStruct((_B, 1), jnp.float32),
      scratch_shapes=[pltpu.VMEM((8, _CP), jnp.float32)],
  )(xc, emb, xc, w0a, w0b_eff, b0_eff, w1, b1r, wo_r, bo_r)


def kernel(x_cat, x_cont, tables, gamma, beta, W0, b0, W1, b1, Wout, bout):
  offs = (jnp.arange(_F, dtype=jnp.int32) * _V)[None, :]
  flat_idx = (x_cat + offs).reshape(_GROUPS, 128)
  tab = tables.reshape(_F * _V, _D)

  emb = _gather(tab, flat_idx).reshape(_B, _E)

  xc = jnp.pad(x_cont, ((0, 0), (0, _CP - _C)))
  w0a = W0[:, :_E]
  w0b = W0[:, _E:]
  w0b_eff = jnp.pad(w0b * gamma[None, :], ((0, 0), (0, _CP - _C)))
  b0_eff = (b0 + w0b @ beta).reshape(1, _H0)
  b1r = b1.reshape(1, _H1)
  wo_r = Wout.reshape(1, _H1)
  bo_r = bout.reshape(1, 1)

  return _mlp(emb, xc, w0a, w0b_eff, b0_eff, W1, b1r, wo_r, bo_r)


# speedup vs baseline: 10.8341x; 4.8444x over previous
import functools

import jax
import jax.numpy as jnp
from jax import lax
from jax.experimental import pallas as pl
from jax.experimental.pallas import tpu as pltpu
from jax.experimental.pallas import tpu_sc as plsc

_B, _F, _V, _D, _C = 16384, 26, 100000, 16, 13
_H0, _H1 = 512, 256
_E = _F * _D
_EPS = 1e-5

_NC, _NS = 2, 16
_NW = _NC * _NS
_RPW = _E // _NW
_HALF = _B // 2


def _make_gather():
  mesh = plsc.VectorSubcoreMesh(core_axis_name="c", subcore_axis_name="s")

  @functools.partial(
      pl.kernel,
      mesh=mesh,
      out_type=jax.ShapeDtypeStruct((_E, _B), jnp.float32),
      scratch_types=[
          pltpu.VMEM((_V,), jnp.float32),
          pltpu.VMEM((_B,), jnp.int32),
          pltpu.VMEM((_HALF,), jnp.float32),
      ],
      compiler_params=pltpu.CompilerParams(use_tc_tiling_on_sc=True,
                                           needs_layout_passes=False),
  )
  def gather_k(m_hbm, idx_hbm, out_hbm, row_v, idx_v, out_v):
    wid = lax.axis_index("s") * _NC + lax.axis_index("c")

    def do_row(t, carry):
      r = wid * _RPW + t
      i = r // _D
      pltpu.sync_copy(idx_hbm.at[i, :], idx_v)
      pltpu.sync_copy(m_hbm.at[r, :], row_v)

      def do_half(h):
        base = h * _HALF

        def gath(k, c):
          o = pl.multiple_of(k * 16, 16)
          idx = idx_v[pl.ds(base + o, 16)]
          out_v[pl.ds(o, 16)] = plsc.load_gather(row_v, [idx])
          return c

        lax.fori_loop(0, _HALF // 16, gath, 0)
        pltpu.sync_copy(out_v, out_hbm.at[r, pl.ds(base, _HALF)])

      do_half(0)
      do_half(1)
      return carry

    lax.fori_loop(0, _RPW, do_row, 0)

  return gather_k


_gather = _make_gather()


def _mlp_body(xcf_ref, emb_ref, xc_ref, w0a_ref, w0b_ref, b0_ref, w1_ref,
              b1_ref, wo_ref, bo_ref, out_ref, stats_ref):
  @pl.when(pl.program_id(0) == 0)
  def _():
    xc = xcf_ref[...]
    m = jnp.mean(xc, axis=1, keepdims=True)
    v = jnp.mean(xc * xc, axis=1, keepdims=True) - m * m
    stats_ref[:, 0:1] = m
    stats_ref[:, 1:2] = lax.rsqrt(v + _EPS)

  m = stats_ref[:, 0:1]
  rstd = stats_ref[:, 1:2]
  x2 = (xc_ref[...] - m) * rstd
  h = lax.dot_general(w0a_ref[...], emb_ref[...],
                      (((1,), (0,)), ((), ())),
                      preferred_element_type=jnp.float32)
  h = h + lax.dot_general(w0b_ref[...], x2,
                          (((1,), (0,)), ((), ())),
                          preferred_element_type=jnp.float32)
  h = jnp.maximum(h + b0_ref[...], 0.0)
  h = lax.dot_general(w1_ref[...], h,
                      (((1,), (0,)), ((), ())),
                      preferred_element_type=jnp.float32)
  h = jnp.maximum(h + b1_ref[...], 0.0)
  out_ref[...] = jnp.sum(h * wo_ref[...], axis=0, keepdims=True) + bo_ref[...]


_BLKN = 1024


def _mlp(embT, xcT, w0a, w0b_eff, b0c, w1, b1c, wo_c, bo_c):
  grid = (_B // _BLKN,)
  return pl.pallas_call(
      _mlp_body,
      grid=grid,
      in_specs=[
          pl.BlockSpec((_C, _B), lambda j: (0, 0)),
          pl.BlockSpec((_E, _BLKN), lambda j: (0, j)),
          pl.BlockSpec((_C, _BLKN), lambda j: (0, j)),
          pl.BlockSpec((_H0, _E), lambda j: (0, 0)),
          pl.BlockSpec((_H0, _C), lambda j: (0, 0)),
          pl.BlockSpec((_H0, 1), lambda j: (0, 0)),
          pl.BlockSpec((_H1, _H0), lambda j: (0, 0)),
          pl.BlockSpec((_H1, 1), lambda j: (0, 0)),
          pl.BlockSpec((_H1, 1), lambda j: (0, 0)),
          pl.BlockSpec((1, 1), lambda j: (0, 0)),
      ],
      out_specs=pl.BlockSpec((1, _BLKN), lambda j: (0, j)),
      out_shape=jax.ShapeDtypeStruct((1, _B), jnp.float32),
      scratch_shapes=[pltpu.VMEM((_C, 128), jnp.float32)],
  )(xcT, embT, xcT, w0a, w0b_eff, b0c, w1, b1c, wo_c, bo_c)


def kernel(x_cat, x_cont, tables, gamma, beta, W0, b0, W1, b1, Wout, bout):
  m_mat = jnp.transpose(tables, (0, 2, 1)).reshape(_E, _V)
  idxT = x_cat.T
  xcT = x_cont.T

  embT = _gather(m_mat, idxT)

  w0a = W0[:, :_E]
  w0b = W0[:, _E:]
  w0b_eff = w0b * gamma[None, :]
  b0c = (b0 + w0b @ beta).reshape(_H0, 1)
  b1c = b1.reshape(_H1, 1)
  wo_c = Wout.reshape(_H1, 1)
  bo_c = bout.reshape(1, 1)

  outT = _mlp(embT, xcT, w0a, w0b_eff, b0c, W1, b1c, wo_c, bo_c)
  return outT.reshape(_B, 1)


# speedup vs baseline: 16.6734x; 1.5390x over previous
import functools

import jax
import jax.numpy as jnp
from jax import lax
from jax.experimental import pallas as pl
from jax.experimental.pallas import tpu as pltpu
from jax.experimental.pallas import tpu_sc as plsc

_B, _F, _V, _D, _C = 16384, 26, 100000, 16, 13
_H0, _H1 = 512, 256
_E = _F * _D
_EPS = 1e-5

_NC, _NS = 2, 16
_NW = _NC * _NS
_RPW = _E // _NW
_HALF = _B // 2


def _make_gather():
  mesh = plsc.VectorSubcoreMesh(core_axis_name="c", subcore_axis_name="s")

  @functools.partial(
      pl.kernel,
      mesh=mesh,
      out_type=jax.ShapeDtypeStruct((_E, _B), jnp.float32),
      scratch_types=[
          pltpu.VMEM((_V,), jnp.float32),
          pltpu.VMEM((_B,), jnp.int32),
          pltpu.VMEM((_HALF,), jnp.float32),
      ],
      compiler_params=pltpu.CompilerParams(use_tc_tiling_on_sc=True,
                                           needs_layout_passes=False),
  )
  def gather_k(m_hbm, idx_hbm, out_hbm, row_v, idx_v, out_v):
    wid = lax.axis_index("s") * _NC + lax.axis_index("c")

    def do_row(t, carry):
      r = wid * _RPW + t
      i = r // _D
      pltpu.sync_copy(idx_hbm.at[i, :], idx_v)
      pltpu.sync_copy(m_hbm.at[r, :], row_v)

      def do_half(h):
        base = h * _HALF

        @plsc.parallel_loop(0, _HALF // 16, unroll=8)
        def _(k):
          o = pl.multiple_of(k * 16, 16)
          idx = idx_v[pl.ds(base + o, 16)]
          out_v[pl.ds(o, 16)] = plsc.load_gather(row_v, [idx])

        pltpu.sync_copy(out_v, out_hbm.at[r, pl.ds(base, _HALF)])

      do_half(0)
      do_half(1)
      return carry

    lax.fori_loop(0, _RPW, do_row, 0)

  return gather_k


_gather = _make_gather()


def _mlp_body(xcf_ref, emb_ref, xc_ref, w0a_ref, w0b_ref, b0_ref, w1_ref,
              b1_ref, wo_ref, bo_ref, out_ref, stats_ref):
  @pl.when(pl.program_id(0) == 0)
  def _():
    xc = xcf_ref[...]
    m = jnp.mean(xc, axis=1, keepdims=True)
    v = jnp.mean(xc * xc, axis=1, keepdims=True) - m * m
    stats_ref[:, 0:1] = m
    stats_ref[:, 1:2] = lax.rsqrt(v + _EPS)

  m = stats_ref[:, 0:1]
  rstd = stats_ref[:, 1:2]
  x2 = (xc_ref[...] - m) * rstd
  h = lax.dot_general(w0a_ref[...], emb_ref[...],
                      (((1,), (0,)), ((), ())),
                      preferred_element_type=jnp.float32)
  h = h + lax.dot_general(w0b_ref[...], x2,
                          (((1,), (0,)), ((), ())),
                          preferred_element_type=jnp.float32)
  h = jnp.maximum(h + b0_ref[...], 0.0)
  h = lax.dot_general(w1_ref[...], h,
                      (((1,), (0,)), ((), ())),
                      preferred_element_type=jnp.float32)
  h = jnp.maximum(h + b1_ref[...], 0.0)
  out_ref[...] = jnp.sum(h * wo_ref[...], axis=0, keepdims=True) + bo_ref[...]


_BLKN = 1024


def _mlp(embT, xcT, w0a, w0b_eff, b0c, w1, b1c, wo_c, bo_c):
  grid = (_B // _BLKN,)
  return pl.pallas_call(
      _mlp_body,
      grid=grid,
      in_specs=[
          pl.BlockSpec((_C, _B), lambda j: (0, 0)),
          pl.BlockSpec((_E, _BLKN), lambda j: (0, j)),
          pl.BlockSpec((_C, _BLKN), lambda j: (0, j)),
          pl.BlockSpec((_H0, _E), lambda j: (0, 0)),
          pl.BlockSpec((_H0, _C), lambda j: (0, 0)),
          pl.BlockSpec((_H0, 1), lambda j: (0, 0)),
          pl.BlockSpec((_H1, _H0), lambda j: (0, 0)),
          pl.BlockSpec((_H1, 1), lambda j: (0, 0)),
          pl.BlockSpec((_H1, 1), lambda j: (0, 0)),
          pl.BlockSpec((1, 1), lambda j: (0, 0)),
      ],
      out_specs=pl.BlockSpec((1, _BLKN), lambda j: (0, j)),
      out_shape=jax.ShapeDtypeStruct((1, _B), jnp.float32),
      scratch_shapes=[pltpu.VMEM((_C, 128), jnp.float32)],
  )(xcT, embT, xcT, w0a, w0b_eff, b0c, w1, b1c, wo_c, bo_c)


def kernel(x_cat, x_cont, tables, gamma, beta, W0, b0, W1, b1, Wout, bout):
  m_mat = jnp.transpose(tables, (0, 2, 1)).reshape(_E, _V)
  idxT = x_cat.T
  xcT = x_cont.T

  embT = _gather(m_mat, idxT)

  w0a = W0[:, :_E]
  w0b = W0[:, _E:]
  w0b_eff = w0b * gamma[None, :]
  b0c = (b0 + w0b @ beta).reshape(_H0, 1)
  b1c = b1.reshape(_H1, 1)
  wo_c = Wout.reshape(_H1, 1)
  bo_c = bout.reshape(1, 1)

  outT = _mlp(embT, xcT, w0a, w0b_eff, b0c, W1, b1c, wo_c, bo_c)
  return outT.reshape(_B, 1)
